# baseline (speedup 1.0000x reference)
import dataclasses
import functools

import jax
import jax.numpy as jnp
from jax import lax
from jax.experimental import pallas as pl
from jax.experimental.pallas import tpu as pltpu
from jax.experimental.pallas import tpu_sc as plsc

D = 256
N_TOK = 16 * 4096
P_PAD = 128
ACC_W = P_PAD * D
NC, NS = 2, 16
NW = NC * NS
CHUNK = N_TOK // NW
TILE = 32
NT = CHUNK // TILE
MAX_SCORE = 2.0
NEG = -1e30
TRASH = (P_PAD - 1) * D


def _sc_body(a_hbm, t_hbm, s_hbm, p_hbm, z_hbm,
             outa_hbm, outt_hbm, outh_hbm,
             acc_a, acc_t, cnt_h, a0, a1, t0, t1, sbuf, pbuf, fbuf,
             sem_a0, sem_a1, sem_t0, sem_t1):
    wid = lax.axis_index("s") * NC + lax.axis_index("c")
    base = wid * CHUNK

    iota16 = lax.iota(jnp.int32, 16)
    zeros16 = jnp.zeros((16,), jnp.float32)
    ones16 = jnp.full((16,), 1.0, jnp.float32)

    pltpu.async_copy(s_hbm.at[pl.ds(base, CHUNK)], sbuf, sem_t0)
    pltpu.async_copy(p_hbm.at[pl.ds(base, CHUNK)], pbuf, sem_t1)
    pltpu.sync_copy(z_hbm, acc_a)
    pltpu.sync_copy(z_hbm, acc_t)
    for k in range(P_PAD // 16):
        cnt_h[pl.ds(16 * k, 16)] = zeros16
    pltpu.make_async_copy(s_hbm.at[pl.ds(base, CHUNK)], sbuf, sem_t0).wait()
    pltpu.make_async_copy(p_hbm.at[pl.ds(base, CHUNK)], pbuf, sem_t1).wait()

    @pl.loop(0, CHUNK // 16)
    def _pre(g):
        pv = pbuf[pl.ds(16 * g, 16)]
        sv = sbuf[pl.ds(16 * g, 16)]
        validm = sv >= 0.0
        fbuf[pl.ds(16 * g, 16)] = jnp.where(validm, pv * D, TRASH)
        highm = jnp.logical_and(validm, sv == MAX_SCORE)
        plsc.store_scatter(cnt_h, [pv], ones16, mask=highm)

    def start(t, abuf, tbuf, sa, st):
        row0 = base + t * TILE
        pltpu.async_copy(a_hbm.at[pl.ds(row0, TILE)], abuf, sa)
        pltpu.async_copy(t_hbm.at[pl.ds(row0, TILE)], tbuf, st)

    def wait(abuf, tbuf, sa, st):
        pltpu.make_async_copy(a_hbm.at[pl.ds(0, TILE)], abuf, sa).wait()
        pltpu.make_async_copy(t_hbm.at[pl.ds(0, TILE)], tbuf, st).wait()

    def accum(t, abuf, tbuf):
        @plsc.parallel_loop(0, TILE, unroll=1)
        def _row(r):
            tok = t * TILE + r
            idxv = jnp.full((16,), tok, jnp.int32)
            fb = plsc.load_gather(fbuf, [idxv]) + iota16
            for c in range(D // 16):
                acc_a_v = acc_a.at[pl.ds(16 * c, ACC_W - 16 * c)]
                acc_t_v = acc_t.at[pl.ds(16 * c, ACC_W - 16 * c)]
                plsc.addupdate_scatter(acc_a_v, [fb], abuf[r, pl.ds(16 * c, 16)])
                plsc.addupdate_scatter(acc_t_v, [fb], tbuf[r, pl.ds(16 * c, 16)])

    start(0, a0, t0, sem_a0, sem_t0)

    @pl.loop(0, NT, step=2)
    def _tile(t):
        start(t + 1, a1, t1, sem_a1, sem_t1)
        wait(a0, t0, sem_a0, sem_t0)
        accum(t, a0, t0)

        @pl.when(t + 2 < NT)
        def _():
            start(t + 2, a0, t0, sem_a0, sem_t0)

        wait(a1, t1, sem_a1, sem_t1)
        accum(t + 1, a1, t1)

    pltpu.sync_copy(acc_a, outa_hbm.at[wid])
    pltpu.sync_copy(acc_t, outt_hbm.at[wid])
    pltpu.sync_copy(cnt_h, outh_hbm.at[wid])


def _sc_partials(a2d, t2d, s1d, p1d, zeros):
    mesh = plsc.VectorSubcoreMesh(core_axis_name="c", subcore_axis_name="s")
    cp = pltpu.CompilerParams()
    if "needs_layout_passes" in pltpu.CompilerParams.__dataclass_fields__:
        cp = dataclasses.replace(cp, needs_layout_passes=False)
    f = pl.kernel(
        _sc_body,
        mesh=mesh,
        compiler_params=cp,
        out_type=[
            jax.ShapeDtypeStruct((NW, ACC_W), jnp.float32),
            jax.ShapeDtypeStruct((NW, ACC_W), jnp.float32),
            jax.ShapeDtypeStruct((NW, P_PAD), jnp.float32),
        ],
        scratch_types=[
            pltpu.VMEM((ACC_W,), jnp.float32),
            pltpu.VMEM((ACC_W,), jnp.float32),
            pltpu.VMEM((P_PAD,), jnp.float32),
            pltpu.VMEM((TILE, D), jnp.float32),
            pltpu.VMEM((TILE, D), jnp.float32),
            pltpu.VMEM((TILE, D), jnp.float32),
            pltpu.VMEM((TILE, D), jnp.float32),
            pltpu.VMEM((CHUNK,), jnp.float32),
            pltpu.VMEM((CHUNK,), jnp.int32),
            pltpu.VMEM((CHUNK,), jnp.int32),
            pltpu.SemaphoreType.DMA,
            pltpu.SemaphoreType.DMA,
            pltpu.SemaphoreType.DMA,
            pltpu.SemaphoreType.DMA,
        ],
    )
    return f(a2d, t2d, s1d, p1d, zeros)


def _tc_body(pa_ref, pt_ref, ch_ref, out_ref):
    sum_a = jnp.sum(pa_ref[...], axis=0)
    sum_t = jnp.sum(pt_ref[...], axis=0)
    cnt_h = jnp.sum(ch_ref[...], axis=0, keepdims=False)
    act_f = (cnt_h > 0.0).astype(jnp.float32).reshape(1, P_PAD)
    n_u = jnp.sum(act_f, keepdims=True)

    def norm(x):
        n = jnp.sqrt(jnp.sum(x * x, axis=1, keepdims=True))
        return x / jnp.maximum(n, 1e-12)

    c_a = norm(sum_a)
    c_t = norm(sum_t)
    dn = (((1,), (1,)), ((), ()))
    cos_at = lax.dot_general(c_a, c_t, dn, preferred_element_type=jnp.float32)
    cos_ta = lax.dot_general(c_t, c_a, dn, preferred_element_type=jnp.float32)
    eye = (lax.broadcasted_iota(jnp.int32, (P_PAD, P_PAD), 0)
           == lax.broadcasted_iota(jnp.int32, (P_PAD, P_PAD), 1))
    diag_act = jnp.logical_and(eye, act_f > 0.0)

    def side(cos):
        m = jnp.where(act_f > 0.0, cos, NEG)
        mx = jnp.max(m, axis=1, keepdims=True)
        lse = jnp.log(jnp.sum(jnp.exp(m - mx), axis=1, keepdims=True)) + mx
        dsum = jnp.sum(jnp.where(diag_act, m - lse, 0.0), keepdims=True)
        return -dsum / jnp.maximum(n_u, 1.0)

    loss = 0.5 * side(cos_at) + 0.5 * side(cos_ta)
    out_ref[...] = jnp.where(n_u >= 2.0, loss, jnp.zeros((1, 1), jnp.float32))


def _tc_finish(parts_a, parts_t, cnts_h):
    return pl.pallas_call(
        _tc_body,
        out_shape=jax.ShapeDtypeStruct((1, 1), jnp.float32),
    )(parts_a, parts_t, cnts_h)


def kernel(audio_features, text_features, scores, phn_ids):
    a2d = audio_features.reshape(-1, D)
    t2d = text_features.reshape(-1, D)
    s1d = scores.reshape(-1).astype(jnp.float32)
    p1d = phn_ids.reshape(-1).astype(jnp.int32)
    zeros = jnp.zeros((ACC_W,), jnp.float32)
    parts_a, parts_t, cnts_h = _sc_partials(a2d, t2d, s1d, p1d, zeros)
    loss = _tc_finish(parts_a.reshape(NW, P_PAD, D),
                      parts_t.reshape(NW, P_PAD, D), cnts_h)
    return loss.reshape(())


# speedup vs baseline: 1.0053x; 1.0053x over previous
import dataclasses
import functools

import jax
import jax.numpy as jnp
from jax import lax
from jax.experimental import pallas as pl
from jax.experimental.pallas import tpu as pltpu
from jax.experimental.pallas import tpu_sc as plsc

D = 256
N_TOK = 16 * 4096
P_PAD = 128
ACC_W = P_PAD * D
NC, NS = 2, 16
NW = NC * NS
CHUNK = N_TOK // NW
TILE = 32
NT = CHUNK // TILE
MAX_SCORE = 2.0
NEG = -1e30
TRASH = (P_PAD - 1) * D


def _sc_body(a_hbm, t_hbm, s_hbm, p_hbm, z_hbm,
             outa_hbm, outt_hbm, outh_hbm,
             acc_a, acc_t, cnt_h, a0, a1, t0, t1, sbuf, pbuf, fbuf,
             sem_a0, sem_a1, sem_t0, sem_t1):
    wid = lax.axis_index("s") * NC + lax.axis_index("c")
    base = wid * CHUNK

    iota16 = lax.iota(jnp.int32, 16)
    zeros16 = jnp.zeros((16,), jnp.float32)
    ones16 = jnp.full((16,), 1.0, jnp.float32)

    pltpu.async_copy(s_hbm.at[pl.ds(base, CHUNK)], sbuf, sem_t0)
    pltpu.async_copy(p_hbm.at[pl.ds(base, CHUNK)], pbuf, sem_t1)
    pltpu.sync_copy(z_hbm, acc_a)
    pltpu.sync_copy(z_hbm, acc_t)
    for k in range(P_PAD // 16):
        cnt_h[pl.ds(16 * k, 16)] = zeros16
    pltpu.make_async_copy(s_hbm.at[pl.ds(base, CHUNK)], sbuf, sem_t0).wait()
    pltpu.make_async_copy(p_hbm.at[pl.ds(base, CHUNK)], pbuf, sem_t1).wait()

    @pl.loop(0, CHUNK // 16)
    def _pre(g):
        pv = pbuf[pl.ds(16 * g, 16)]
        sv = sbuf[pl.ds(16 * g, 16)]
        validm = sv >= 0.0
        fbuf[pl.ds(16 * g, 16)] = jnp.where(validm, pv * D, TRASH)
        highm = jnp.logical_and(validm, sv == MAX_SCORE)
        plsc.store_scatter(cnt_h, [pv], ones16, mask=highm)

    def start(t, abuf, tbuf, sa, st):
        row0 = base + t * TILE
        pltpu.async_copy(a_hbm.at[pl.ds(row0, TILE)], abuf, sa)
        pltpu.async_copy(t_hbm.at[pl.ds(row0, TILE)], tbuf, st)

    def wait(abuf, tbuf, sa, st):
        pltpu.make_async_copy(a_hbm.at[pl.ds(0, TILE)], abuf, sa).wait()
        pltpu.make_async_copy(t_hbm.at[pl.ds(0, TILE)], tbuf, st).wait()

    def accum(t, abuf, tbuf):
        @plsc.parallel_loop(0, TILE, unroll=2)
        def _row(r):
            tok = t * TILE + r
            idxv = jnp.full((16,), tok, jnp.int32)
            fb = plsc.load_gather(fbuf, [idxv]) + iota16
            for c in range(D // 16):
                acc_a_v = acc_a.at[pl.ds(16 * c, ACC_W - 16 * c)]
                acc_t_v = acc_t.at[pl.ds(16 * c, ACC_W - 16 * c)]
                plsc.addupdate_scatter(acc_a_v, [fb], abuf[r, pl.ds(16 * c, 16)])
                plsc.addupdate_scatter(acc_t_v, [fb], tbuf[r, pl.ds(16 * c, 16)])

    start(0, a0, t0, sem_a0, sem_t0)

    @pl.loop(0, NT, step=2)
    def _tile(t):
        start(t + 1, a1, t1, sem_a1, sem_t1)
        wait(a0, t0, sem_a0, sem_t0)
        accum(t, a0, t0)

        @pl.when(t + 2 < NT)
        def _():
            start(t + 2, a0, t0, sem_a0, sem_t0)

        wait(a1, t1, sem_a1, sem_t1)
        accum(t + 1, a1, t1)

    pltpu.sync_copy(acc_a, outa_hbm.at[wid])
    pltpu.sync_copy(acc_t, outt_hbm.at[wid])
    pltpu.sync_copy(cnt_h, outh_hbm.at[wid])


def _sc_partials(a2d, t2d, s1d, p1d, zeros):
    mesh = plsc.VectorSubcoreMesh(core_axis_name="c", subcore_axis_name="s")
    cp = pltpu.CompilerParams()
    if "needs_layout_passes" in pltpu.CompilerParams.__dataclass_fields__:
        cp = dataclasses.replace(cp, needs_layout_passes=False)
    f = pl.kernel(
        _sc_body,
        mesh=mesh,
        compiler_params=cp,
        out_type=[
            jax.ShapeDtypeStruct((NW, ACC_W), jnp.float32),
            jax.ShapeDtypeStruct((NW, ACC_W), jnp.float32),
            jax.ShapeDtypeStruct((NW, P_PAD), jnp.float32),
        ],
        scratch_types=[
            pltpu.VMEM((ACC_W,), jnp.float32),
            pltpu.VMEM((ACC_W,), jnp.float32),
            pltpu.VMEM((P_PAD,), jnp.float32),
            pltpu.VMEM((TILE, D), jnp.float32),
            pltpu.VMEM((TILE, D), jnp.float32),
            pltpu.VMEM((TILE, D), jnp.float32),
            pltpu.VMEM((TILE, D), jnp.float32),
            pltpu.VMEM((CHUNK,), jnp.float32),
            pltpu.VMEM((CHUNK,), jnp.int32),
            pltpu.VMEM((CHUNK,), jnp.int32),
            pltpu.SemaphoreType.DMA,
            pltpu.SemaphoreType.DMA,
            pltpu.SemaphoreType.DMA,
            pltpu.SemaphoreType.DMA,
        ],
    )
    return f(a2d, t2d, s1d, p1d, zeros)


def _tc_body(pa_ref, pt_ref, ch_ref, out_ref):
    sum_a = jnp.sum(pa_ref[...], axis=0)
    sum_t = jnp.sum(pt_ref[...], axis=0)
    cnt_h = jnp.sum(ch_ref[...], axis=0, keepdims=False)
    act_f = (cnt_h > 0.0).astype(jnp.float32).reshape(1, P_PAD)
    n_u = jnp.sum(act_f, keepdims=True)

    def norm(x):
        n = jnp.sqrt(jnp.sum(x * x, axis=1, keepdims=True))
        return x / jnp.maximum(n, 1e-12)

    c_a = norm(sum_a)
    c_t = norm(sum_t)
    dn = (((1,), (1,)), ((), ()))
    cos_at = lax.dot_general(c_a, c_t, dn, preferred_element_type=jnp.float32)
    cos_ta = lax.dot_general(c_t, c_a, dn, preferred_element_type=jnp.float32)
    eye = (lax.broadcasted_iota(jnp.int32, (P_PAD, P_PAD), 0)
           == lax.broadcasted_iota(jnp.int32, (P_PAD, P_PAD), 1))
    diag_act = jnp.logical_and(eye, act_f > 0.0)

    def side(cos):
        m = jnp.where(act_f > 0.0, cos, NEG)
        mx = jnp.max(m, axis=1, keepdims=True)
        lse = jnp.log(jnp.sum(jnp.exp(m - mx), axis=1, keepdims=True)) + mx
        dsum = jnp.sum(jnp.where(diag_act, m - lse, 0.0), keepdims=True)
        return -dsum / jnp.maximum(n_u, 1.0)

    loss = 0.5 * side(cos_at) + 0.5 * side(cos_ta)
    out_ref[...] = jnp.where(n_u >= 2.0, loss, jnp.zeros((1, 1), jnp.float32))


def _tc_finish(parts_a, parts_t, cnts_h):
    return pl.pallas_call(
        _tc_body,
        out_shape=jax.ShapeDtypeStruct((1, 1), jnp.float32),
    )(parts_a, parts_t, cnts_h)


def kernel(audio_features, text_features, scores, phn_ids):
    a2d = audio_features.reshape(-1, D)
    t2d = text_features.reshape(-1, D)
    s1d = scores.reshape(-1).astype(jnp.float32)
    p1d = phn_ids.reshape(-1).astype(jnp.int32)
    zeros = jnp.zeros((ACC_W,), jnp.float32)
    parts_a, parts_t, cnts_h = _sc_partials(a2d, t2d, s1d, p1d, zeros)
    loss = _tc_finish(parts_a.reshape(NW, P_PAD, D),
                      parts_t.reshape(NW, P_PAD, D), cnts_h)
    return loss.reshape(())


# speedup vs baseline: 1.0075x; 1.0022x over previous
import dataclasses
import functools

import jax
import jax.numpy as jnp
from jax import lax
from jax.experimental import pallas as pl
from jax.experimental.pallas import tpu as pltpu
from jax.experimental.pallas import tpu_sc as plsc

D = 256
N_TOK = 16 * 4096
P_PAD = 128
ACC_W = P_PAD * D
NC, NS = 2, 16
NW = NC * NS
CHUNK = N_TOK // NW
TILE = 32
NT = CHUNK // TILE
MAX_SCORE = 2.0
NEG = -1e30
TRASH = (P_PAD - 1) * D


def _sc_body(a_hbm, t_hbm, s_hbm, p_hbm, z_hbm,
             outa_hbm, outt_hbm, outh_hbm,
             acc_a, acc_t, cnt_h, a0, a1, t0, t1, sbuf, pbuf, fbuf,
             sem_a0, sem_a1, sem_t0, sem_t1):
    wid = lax.axis_index("s") * NC + lax.axis_index("c")
    base = wid * CHUNK

    iota16 = lax.iota(jnp.int32, 16)
    zeros16 = jnp.zeros((16,), jnp.float32)
    ones16 = jnp.full((16,), 1.0, jnp.float32)

    pltpu.async_copy(s_hbm.at[pl.ds(base, CHUNK)], sbuf, sem_t0)
    pltpu.async_copy(p_hbm.at[pl.ds(base, CHUNK)], pbuf, sem_t1)
    pltpu.sync_copy(z_hbm, acc_a)
    pltpu.sync_copy(z_hbm, acc_t)
    for k in range(P_PAD // 16):
        cnt_h[pl.ds(16 * k, 16)] = zeros16
    pltpu.make_async_copy(s_hbm.at[pl.ds(base, CHUNK)], sbuf, sem_t0).wait()
    pltpu.make_async_copy(p_hbm.at[pl.ds(base, CHUNK)], pbuf, sem_t1).wait()

    @pl.loop(0, CHUNK // 16)
    def _pre(g):
        pv = pbuf[pl.ds(16 * g, 16)]
        sv = sbuf[pl.ds(16 * g, 16)]
        validm = sv >= 0.0
        fbuf[pl.ds(16 * g, 16)] = jnp.where(validm, pv * D, TRASH)
        highm = jnp.logical_and(validm, sv == MAX_SCORE)
        plsc.store_scatter(cnt_h, [pv], ones16, mask=highm)

    def start(t, abuf, tbuf, sa, st):
        row0 = base + t * TILE
        pltpu.async_copy(a_hbm.at[pl.ds(row0, TILE)], abuf, sa)
        pltpu.async_copy(t_hbm.at[pl.ds(row0, TILE)], tbuf, st)

    def wait(abuf, tbuf, sa, st):
        pltpu.make_async_copy(a_hbm.at[pl.ds(0, TILE)], abuf, sa).wait()
        pltpu.make_async_copy(t_hbm.at[pl.ds(0, TILE)], tbuf, st).wait()

    def accum(t, abuf, tbuf):
        @plsc.parallel_loop(0, TILE, unroll=2)
        def _row_a(r):
            tok = t * TILE + r
            idxv = jnp.full((16,), tok, jnp.int32)
            fb = plsc.load_gather(fbuf, [idxv]) + iota16
            for c in range(D // 16):
                acc_a_v = acc_a.at[pl.ds(16 * c, ACC_W - 16 * c)]
                plsc.addupdate_scatter(acc_a_v, [fb], abuf[r, pl.ds(16 * c, 16)])

        @plsc.parallel_loop(0, TILE, unroll=2)
        def _row_t(r):
            tok = t * TILE + r
            idxv = jnp.full((16,), tok, jnp.int32)
            fb = plsc.load_gather(fbuf, [idxv]) + iota16
            for c in range(D // 16):
                acc_t_v = acc_t.at[pl.ds(16 * c, ACC_W - 16 * c)]
                plsc.addupdate_scatter(acc_t_v, [fb], tbuf[r, pl.ds(16 * c, 16)])

    start(0, a0, t0, sem_a0, sem_t0)

    @pl.loop(0, NT, step=2)
    def _tile(t):
        start(t + 1, a1, t1, sem_a1, sem_t1)
        wait(a0, t0, sem_a0, sem_t0)
        accum(t, a0, t0)

        @pl.when(t + 2 < NT)
        def _():
            start(t + 2, a0, t0, sem_a0, sem_t0)

        wait(a1, t1, sem_a1, sem_t1)
        accum(t + 1, a1, t1)

    pltpu.sync_copy(acc_a, outa_hbm.at[wid])
    pltpu.sync_copy(acc_t, outt_hbm.at[wid])
    pltpu.sync_copy(cnt_h, outh_hbm.at[wid])


def _sc_partials(a2d, t2d, s1d, p1d, zeros):
    mesh = plsc.VectorSubcoreMesh(core_axis_name="c", subcore_axis_name="s")
    cp = pltpu.CompilerParams()
    if "needs_layout_passes" in pltpu.CompilerParams.__dataclass_fields__:
        cp = dataclasses.replace(cp, needs_layout_passes=False)
    f = pl.kernel(
        _sc_body,
        mesh=mesh,
        compiler_params=cp,
        out_type=[
            jax.ShapeDtypeStruct((NW, ACC_W), jnp.float32),
            jax.ShapeDtypeStruct((NW, ACC_W), jnp.float32),
            jax.ShapeDtypeStruct((NW, P_PAD), jnp.float32),
        ],
        scratch_types=[
            pltpu.VMEM((ACC_W,), jnp.float32),
            pltpu.VMEM((ACC_W,), jnp.float32),
            pltpu.VMEM((P_PAD,), jnp.float32),
            pltpu.VMEM((TILE, D), jnp.float32),
            pltpu.VMEM((TILE, D), jnp.float32),
            pltpu.VMEM((TILE, D), jnp.float32),
            pltpu.VMEM((TILE, D), jnp.float32),
            pltpu.VMEM((CHUNK,), jnp.float32),
            pltpu.VMEM((CHUNK,), jnp.int32),
            pltpu.VMEM((CHUNK,), jnp.int32),
            pltpu.SemaphoreType.DMA,
            pltpu.SemaphoreType.DMA,
            pltpu.SemaphoreType.DMA,
            pltpu.SemaphoreType.DMA,
        ],
    )
    return f(a2d, t2d, s1d, p1d, zeros)


def _tc_body(pa_ref, pt_ref, ch_ref, out_ref):
    sum_a = jnp.sum(pa_ref[...], axis=0)
    sum_t = jnp.sum(pt_ref[...], axis=0)
    cnt_h = jnp.sum(ch_ref[...], axis=0, keepdims=False)
    act_f = (cnt_h > 0.0).astype(jnp.float32).reshape(1, P_PAD)
    n_u = jnp.sum(act_f, keepdims=True)

    def norm(x):
        n = jnp.sqrt(jnp.sum(x * x, axis=1, keepdims=True))
        return x / jnp.maximum(n, 1e-12)

    c_a = norm(sum_a)
    c_t = norm(sum_t)
    dn = (((1,), (1,)), ((), ()))
    cos_at = lax.dot_general(c_a, c_t, dn, preferred_element_type=jnp.float32)
    cos_ta = lax.dot_general(c_t, c_a, dn, preferred_element_type=jnp.float32)
    eye = (lax.broadcasted_iota(jnp.int32, (P_PAD, P_PAD), 0)
           == lax.broadcasted_iota(jnp.int32, (P_PAD, P_PAD), 1))
    diag_act = jnp.logical_and(eye, act_f > 0.0)

    def side(cos):
        m = jnp.where(act_f > 0.0, cos, NEG)
        mx = jnp.max(m, axis=1, keepdims=True)
        lse = jnp.log(jnp.sum(jnp.exp(m - mx), axis=1, keepdims=True)) + mx
        dsum = jnp.sum(jnp.where(diag_act, m - lse, 0.0), keepdims=True)
        return -dsum / jnp.maximum(n_u, 1.0)

    loss = 0.5 * side(cos_at) + 0.5 * side(cos_ta)
    out_ref[...] = jnp.where(n_u >= 2.0, loss, jnp.zeros((1, 1), jnp.float32))


def _tc_finish(parts_a, parts_t, cnts_h):
    return pl.pallas_call(
        _tc_body,
        out_shape=jax.ShapeDtypeStruct((1, 1), jnp.float32),
    )(parts_a, parts_t, cnts_h)


def kernel(audio_features, text_features, scores, phn_ids):
    a2d = audio_features.reshape(-1, D)
    t2d = text_features.reshape(-1, D)
    s1d = scores.reshape(-1).astype(jnp.float32)
    p1d = phn_ids.reshape(-1).astype(jnp.int32)
    zeros = jnp.zeros((ACC_W,), jnp.float32)
    parts_a, parts_t, cnts_h = _sc_partials(a2d, t2d, s1d, p1d, zeros)
    loss = _tc_finish(parts_a.reshape(NW, P_PAD, D),
                      parts_t.reshape(NW, P_PAD, D), cnts_h)
    return loss.reshape(())
